# Initial kernel scaffold; baseline (speedup 1.0000x reference)
import jax
import jax.numpy as jnp
from jax.experimental import pallas as pl


def kernel(z, edge_index, W, b):
    raise NotImplementedError("write your pallas kernel here")


# speedup vs baseline: 6.2398x; 6.2398x over previous
import functools

import jax
import jax.numpy as jnp
from jax import lax
from jax.experimental import pallas as pl
from jax.experimental.pallas import tpu as pltpu
from jax.experimental.pallas import tpu_sc as plsc

N_NODES = 10000
N_EDGES = 320000
N_Z = 128
EDGE_DIM = 16

_info = plsc.get_sparse_core_info()
NC, NS = _info.num_cores, _info.num_subcores
NW = NC * NS
EDGES_PER_W = N_EDGES // NW
CHUNK = 2000
N_CHUNKS = EDGES_PER_W // CHUNK


def _tables_body(z_ref, w1_ref, w2_ref, b_ref, t1_ref, t2_ref):
    z = z_ref[...]
    dn = (((1,), (1,)), ((), ()))
    t1_ref[...] = (
        jax.lax.dot_general(z, w1_ref[...], dn, preferred_element_type=jnp.float32)
        + b_ref[...]
    )
    t2_ref[...] = jax.lax.dot_general(
        z, w2_ref[...], dn, preferred_element_type=jnp.float32
    )


def _make_tables(z, W1, W2, b2d):
    return pl.pallas_call(
        _tables_body,
        out_shape=[
            jax.ShapeDtypeStruct((N_NODES, EDGE_DIM), jnp.float32),
            jax.ShapeDtypeStruct((N_NODES, EDGE_DIM), jnp.float32),
        ],
    )(z, W1, W2, b2d)


@functools.partial(
    pl.kernel,
    out_type=jax.ShapeDtypeStruct((N_EDGES, EDGE_DIM), jnp.float32),
    mesh=plsc.VectorSubcoreMesh(core_axis_name="c", subcore_axis_name="s"),
    compiler_params=pltpu.CompilerParams(use_tc_tiling_on_sc=False),
    scratch_types=[
        pltpu.VMEM((CHUNK,), jnp.int32),
        pltpu.VMEM((CHUNK,), jnp.int32),
        pltpu.VMEM((CHUNK, EDGE_DIM), jnp.float32),
        pltpu.VMEM((CHUNK, EDGE_DIM), jnp.float32),
        pltpu.SemaphoreType.DMA,
        pltpu.SemaphoreType.DMA,
    ],
)
def _edge_gather_add(t1_hbm, t2_hbm, src_hbm, dst_hbm, out_hbm,
                     idx1, idx2, r1, r2, sem1, sem2):
    wid = lax.axis_index("s") * NC + lax.axis_index("c")
    base = wid * EDGES_PER_W

    def chunk_body(c, carry):
        off = base + c * CHUNK
        pltpu.sync_copy(src_hbm.at[pl.ds(off, CHUNK)], idx1)
        pltpu.sync_copy(dst_hbm.at[pl.ds(off, CHUNK)], idx2)
        cp1 = pltpu.async_copy(t1_hbm.at[idx1], r1, sem1)
        cp2 = pltpu.async_copy(t2_hbm.at[idx2], r2, sem2)
        cp1.wait()
        cp2.wait()

        def row_body(i, c2):
            r1[i, :] = r1[i, :] + r2[i, :]
            return c2

        lax.fori_loop(0, CHUNK, row_body, 0, unroll=8)
        pltpu.sync_copy(r1, out_hbm.at[pl.ds(off, CHUNK)])
        return carry

    lax.fori_loop(0, N_CHUNKS, chunk_body, 0)


def kernel(z, edge_index, W, b):
    edge_index = edge_index.astype(jnp.int32)
    W1 = W[:, :N_Z]
    W2 = W[:, N_Z:]
    t1, t2 = _make_tables(z, W1, W2, b.reshape(1, EDGE_DIM))
    return _edge_gather_add(t1, t2, edge_index[0], edge_index[1])


# speedup vs baseline: 6.6684x; 1.0687x over previous
import functools

import jax
import jax.numpy as jnp
from jax import lax
from jax.experimental import pallas as pl
from jax.experimental.pallas import tpu as pltpu
from jax.experimental.pallas import tpu_sc as plsc

N_NODES = 10000
N_EDGES = 320000
N_Z = 128
EDGE_DIM = 16

_info = plsc.get_sparse_core_info()
NC, NS = _info.num_cores, _info.num_subcores
NW = NC * NS
EDGES_PER_W = N_EDGES // NW
CHUNK = 1000
N_CHUNKS = EDGES_PER_W // CHUNK


def _tables_body(z_ref, w1_ref, w2_ref, b_ref, t1_ref, t2_ref):
    z = z_ref[...]
    dn = (((1,), (1,)), ((), ()))
    t1_ref[...] = (
        jax.lax.dot_general(z, w1_ref[...], dn, preferred_element_type=jnp.float32)
        + b_ref[...]
    )
    t2_ref[...] = jax.lax.dot_general(
        z, w2_ref[...], dn, preferred_element_type=jnp.float32
    )


def _make_tables(z, W1, W2, b2d):
    return pl.pallas_call(
        _tables_body,
        out_shape=[
            jax.ShapeDtypeStruct((N_NODES, EDGE_DIM), jnp.float32),
            jax.ShapeDtypeStruct((N_NODES, EDGE_DIM), jnp.float32),
        ],
    )(z, W1, W2, b2d)


@functools.partial(
    pl.kernel,
    out_type=jax.ShapeDtypeStruct((N_EDGES, EDGE_DIM), jnp.float32),
    mesh=plsc.VectorSubcoreMesh(core_axis_name="c", subcore_axis_name="s"),
    compiler_params=pltpu.CompilerParams(use_tc_tiling_on_sc=False),
    scratch_types=[
        pltpu.VMEM((2, CHUNK), jnp.int32),
        pltpu.VMEM((2, CHUNK), jnp.int32),
        pltpu.VMEM((CHUNK, EDGE_DIM), jnp.float32),
        pltpu.VMEM((CHUNK, EDGE_DIM), jnp.float32),
        pltpu.VMEM((CHUNK, EDGE_DIM), jnp.float32),
        pltpu.VMEM((CHUNK, EDGE_DIM), jnp.float32),
        pltpu.SemaphoreType.DMA,
        pltpu.SemaphoreType.DMA,
        pltpu.SemaphoreType.DMA,
        pltpu.SemaphoreType.DMA,
    ],
)
def _edge_gather_add(t1_hbm, t2_hbm, src_hbm, dst_hbm, out_hbm,
                     idx1, idx2, r1a, r2a, r1b, r2b,
                     sem1a, sem2a, sem1b, sem2b):
    wid = lax.axis_index("s") * NC + lax.axis_index("c")
    base = wid * EDGES_PER_W
    r1 = (r1a, r1b)
    r2 = (r2a, r2b)
    sems = ((sem1a, sem2a), (sem1b, sem2b))

    def issue(c, buf):
        off = base + c * CHUNK
        pltpu.sync_copy(src_hbm.at[pl.ds(off, CHUNK)], idx1.at[buf])
        pltpu.sync_copy(dst_hbm.at[pl.ds(off, CHUNK)], idx2.at[buf])
        cp1 = pltpu.async_copy(t1_hbm.at[idx1.at[buf]], r1[buf], sems[buf][0])
        cp2 = pltpu.async_copy(t2_hbm.at[idx2.at[buf]], r2[buf], sems[buf][1])
        return cp1, cp2

    pending = issue(0, 0)
    for c in range(N_CHUNKS):
        buf = c % 2
        if c + 1 < N_CHUNKS:
            nxt = issue(c + 1, (c + 1) % 2)
        pending[0].wait()
        pending[1].wait()

        def row_body(i, carry, a=r1[buf], b=r2[buf]):
            a[i, :] = a[i, :] + b[i, :]
            return carry

        lax.fori_loop(0, CHUNK, row_body, 0, unroll=8)
        pltpu.sync_copy(r1[buf], out_hbm.at[pl.ds(base + c * CHUNK, CHUNK)])
        if c + 1 < N_CHUNKS:
            pending = nxt


def kernel(z, edge_index, W, b):
    edge_index = edge_index.astype(jnp.int32)
    W1 = W[:, :N_Z]
    W2 = W[:, N_Z:]
    t1, t2 = _make_tables(z, W1, W2, b.reshape(1, EDGE_DIM))
    return _edge_gather_add(t1, t2, edge_index[0], edge_index[1])
